# baseline (speedup 1.0000x reference)
import functools

import jax
import jax.numpy as jnp
from jax import lax
from jax.experimental import pallas as pl
from jax.experimental.pallas import tpu as pltpu
from jax.experimental.pallas import tpu_sc as plsc

VOCAB = 1000000
D_MODEL = 64
LANES = 16
NUM_CORES = 2
NUM_SUBCORES = 16
NUM_WORKERS = NUM_CORES * NUM_SUBCORES
VBLOCK = 128
N_FULL_BLOCKS = VOCAB // VBLOCK
TAIL_V = VOCAB - N_FULL_BLOCKS * VBLOCK
SCR_ROWS = VOCAB // 2
GROUP = 128
SCALE = 8.0
NBUF = 3
LEAD = 2


def _iota16():
    return lax.iota(jnp.int32, LANES)


@jax.jit
def _format(tt, tail_scr):
    mesh = plsc.VectorSubcoreMesh(core_axis_name="c", subcore_axis_name="s")
    n_steps = (N_FULL_BLOCKS + NUM_WORKERS - 1) // NUM_WORKERS

    @functools.partial(
        pl.kernel,
        mesh=mesh,
        out_type=jax.ShapeDtypeStruct((SCR_ROWS, 2 * D_MODEL), jnp.float32),
        scratch_types=[
            tuple(pltpu.VMEM((D_MODEL, VBLOCK), jnp.float32) for _ in range(NBUF)),
            tuple(
                pltpu.VMEM((VBLOCK // 2, 2 * D_MODEL), jnp.float32)
                for _ in range(NBUF)
            ),
            tuple(pltpu.SemaphoreType.DMA for _ in range(NBUF)),
            tuple(pltpu.SemaphoreType.DMA for _ in range(NBUF)),
        ],
        compiler_params=pltpu.CompilerParams(use_tc_tiling_on_sc=True, needs_layout_passes=False),
    )
    def k(tt_hbm, tail_hbm, scr_hbm, tbufs, obufs, isems, osems):
        wid = lax.axis_index("s") * NUM_CORES + lax.axis_index("c")
        iota = _iota16()

        def block_id(i):
            return wid + NUM_WORKERS * i

        def fire_load(i, b):
            pltpu.async_copy(
                tt_hbm.at[pl.ds(0, D_MODEL), pl.ds(block_id(i) * VBLOCK, VBLOCK)],
                tbufs[b],
                isems[b],
            )

        for i0 in range(LEAD):
            fire_load(i0, i0)

        def ring_body(p, carry):
            for b in range(NBUF):
                i = NBUF * p + b

                @pl.when(block_id(i) < N_FULL_BLOCKS)
                def _():
                    pltpu.make_async_copy(
                        tt_hbm.at[pl.ds(0, D_MODEL), pl.ds(0, VBLOCK)],
                        tbufs[b],
                        isems[b],
                    ).wait()

                    @pl.when(i >= NBUF)
                    def _():
                        pltpu.make_async_copy(
                            obufs[b],
                            scr_hbm.at[pl.ds(0, VBLOCK // 2)],
                            osems[b],
                        ).wait()

                    @plsc.parallel_loop(0, VBLOCK // 2, unroll=4)
                    def _(j):
                        for kk in range(8):
                            d_idx = iota + (16 * (kk % 4))
                            v_idx = jnp.zeros((LANES,), jnp.int32) + (
                                2 * j + (kk // 4)
                            )
                            vec = plsc.load_gather(tbufs[b], [d_idx, v_idx])
                            obufs[b][j, pl.ds(16 * kk, 16)] = vec * SCALE

                    pltpu.async_copy(
                        obufs[b],
                        scr_hbm.at[pl.ds(block_id(i) * (VBLOCK // 2), VBLOCK // 2)],
                        osems[b],
                    )

                    @pl.when(block_id(i + LEAD) < N_FULL_BLOCKS)
                    def _():
                        fire_load(i + LEAD, (b + LEAD) % NBUF)

            return carry

        lax.fori_loop(0, (n_steps + NBUF - 1) // NBUF, ring_body, 0)
        for b in range(NBUF):
            pltpu.make_async_copy(
                obufs[b], scr_hbm.at[pl.ds(0, VBLOCK // 2)], osems[b]
            ).wait()

        @pl.when(wid == NUM_WORKERS - 1)
        def _():
            pltpu.sync_copy(tail_hbm, obufs[0].at[pl.ds(0, TAIL_V // 2)])
            pltpu.sync_copy(
                obufs[0].at[pl.ds(0, TAIL_V // 2)],
                scr_hbm.at[pl.ds(N_FULL_BLOCKS * (VBLOCK // 2), TAIL_V // 2)],
            )

    return k(tt, tail_scr)


@jax.jit
def _gather(xg, scr2):
    mesh = plsc.VectorSubcoreMesh(core_axis_name="c", subcore_axis_name="s")
    n_seq = xg.shape[1]

    @functools.partial(
        pl.kernel,
        mesh=mesh,
        out_type=jax.ShapeDtypeStruct(
            (n_seq, D_MODEL // 8, NUM_WORKERS, 8, GROUP), jnp.float32
        ),
        scratch_types=[
            pltpu.VMEM((n_seq, GROUP), jnp.int32),
            tuple(pltpu.VMEM((GROUP, D_MODEL), jnp.float32) for _ in range(NBUF)),
            tuple(
                pltpu.VMEM((D_MODEL // 8, 8, GROUP), jnp.float32)
                for _ in range(NBUF)
            ),
            tuple(pltpu.SemaphoreType.DMA for _ in range(NBUF)),
            tuple(pltpu.SemaphoreType.DMA for _ in range(NBUF)),
        ],
        compiler_params=pltpu.CompilerParams(use_tc_tiling_on_sc=False, needs_layout_passes=False),
    )
    def k(x_hbm, tab_hbm, out_hbm, idx_v, gbufs, obufs, gsems, ssems):
        wid = lax.axis_index("s") * NUM_CORES + lax.axis_index("c")
        iota = _iota16()
        pltpu.sync_copy(x_hbm.at[wid], idx_v)

        def fire_gather(s, b):
            pltpu.async_copy(tab_hbm.at[idx_v.at[s]], gbufs[b], gsems[b])

        for s0 in range(LEAD):
            fire_gather(s0, s0)

        def ring_body(p, carry):
            for b in range(NBUF):
                s = NBUF * p + b

                @pl.when(s < n_seq)
                def _():
                    pltpu.make_async_copy(
                        tab_hbm.at[idx_v.at[0]], gbufs[b], gsems[b]
                    ).wait()

                    @pl.when(s >= NBUF)
                    def _():
                        pltpu.make_async_copy(
                            obufs[b],
                            out_hbm.at[0, pl.ds(0, D_MODEL // 8), 0],
                            ssems[b],
                        ).wait()

                    @plsc.parallel_loop(0, GROUP // LANES, unroll=2)
                    def _(kb):
                        rows = kb * LANES + iota
                        for dt in range(D_MODEL // 8):
                            for di in range(8):
                                cols = jnp.zeros((LANES,), jnp.int32) + (dt * 8 + di)
                                vec = plsc.load_gather(gbufs[b], [rows, cols])
                                obufs[b][dt, di, pl.ds(kb * LANES, LANES)] = vec

                    pltpu.async_copy(
                        obufs[b],
                        out_hbm.at[s, pl.ds(0, D_MODEL // 8), wid],
                        ssems[b],
                    )

                    @pl.when(s + LEAD < n_seq)
                    def _():
                        fire_gather(s + LEAD, (b + LEAD) % NBUF)

            return carry

        lax.fori_loop(0, (n_seq + NBUF - 1) // NBUF, ring_body, 0)
        for b in range(NBUF):
            pltpu.make_async_copy(
                obufs[b],
                out_hbm.at[0, pl.ds(0, D_MODEL // 8), 0],
                ssems[b],
            ).wait()

    return k(xg, scr2)


def kernel(x, embedding_weight):
    batch, seq = x.shape
    tt = jnp.transpose(embedding_weight)
    tail_scr = (jnp.transpose(tt[:, N_FULL_BLOCKS * VBLOCK :]) * SCALE).reshape(
        TAIL_V // 2, 2 * D_MODEL
    )
    scr = _format(tt, tail_scr)
    scr2 = scr.reshape(VOCAB, D_MODEL)
    xg = (
        jnp.transpose(x.astype(jnp.int32))
        .reshape(seq, NUM_WORKERS, GROUP)
        .transpose(1, 0, 2)
    )
    out5 = _gather(xg, scr2)
    return jnp.transpose(out5, (2, 4, 0, 1, 3)).reshape(batch, seq, D_MODEL)


# speedup vs baseline: 1.0526x; 1.0526x over previous
import functools

import jax
import jax.numpy as jnp
from jax import lax
from jax.experimental import pallas as pl
from jax.experimental.pallas import tpu as pltpu
from jax.experimental.pallas import tpu_sc as plsc

VOCAB = 1000000
D_MODEL = 64
LANES = 16
NUM_CORES = 2
NUM_SUBCORES = 16
NUM_WORKERS = NUM_CORES * NUM_SUBCORES
VBLOCK = 256
N_FULL_BLOCKS = VOCAB // VBLOCK
TAIL_V = VOCAB - N_FULL_BLOCKS * VBLOCK
GROUP = 128
SCALE = 8.0
NBUF = 3
LEAD = 2


def _iota16():
    return lax.iota(jnp.int32, LANES)


@jax.jit
def _format(tt, tail_scr):
    mesh = plsc.VectorSubcoreMesh(core_axis_name="c", subcore_axis_name="s")
    n_steps = (N_FULL_BLOCKS + NUM_WORKERS - 1) // NUM_WORKERS

    @functools.partial(
        pl.kernel,
        mesh=mesh,
        out_type=jax.ShapeDtypeStruct((VOCAB * D_MODEL,), jnp.float32),
        scratch_types=[
            tuple(pltpu.VMEM((D_MODEL, VBLOCK), jnp.float32) for _ in range(NBUF)),
            tuple(pltpu.VMEM((VBLOCK * D_MODEL,), jnp.float32) for _ in range(NBUF)),
            tuple(pltpu.SemaphoreType.DMA for _ in range(NBUF)),
            tuple(pltpu.SemaphoreType.DMA for _ in range(NBUF)),
        ],
        compiler_params=pltpu.CompilerParams(
            use_tc_tiling_on_sc=True, needs_layout_passes=False
        ),
    )
    def k(tt_hbm, tail_hbm, scr_hbm, tbufs, obufs, isems, osems):
        wid = lax.axis_index("s") * NUM_CORES + lax.axis_index("c")
        iota = _iota16()
        pat = iota * D_MODEL

        def block_id(i):
            return wid + NUM_WORKERS * i

        def fire_load(i, b):
            pltpu.async_copy(
                tt_hbm.at[pl.ds(0, D_MODEL), pl.ds(block_id(i) * VBLOCK, VBLOCK)],
                tbufs[b],
                isems[b],
            )

        for i0 in range(LEAD):
            fire_load(i0, i0)

        def ring_body(p, carry):
            for b in range(NBUF):
                i = NBUF * p + b

                @pl.when(block_id(i) < N_FULL_BLOCKS)
                def _():
                    pltpu.make_async_copy(
                        tt_hbm.at[pl.ds(0, D_MODEL), pl.ds(0, VBLOCK)],
                        tbufs[b],
                        isems[b],
                    ).wait()

                    @pl.when(i >= NBUF)
                    def _():
                        pltpu.make_async_copy(
                            obufs[b], scr_hbm.at[pl.ds(0, VBLOCK * D_MODEL)], osems[b]
                        ).wait()

                    @plsc.parallel_loop(0, D_MODEL, unroll=2)
                    def _(d):
                        for kk in range(VBLOCK // LANES):
                            vec = tbufs[b][d, pl.ds(kk * LANES, LANES)] * SCALE
                            idx = pat + (kk * LANES * D_MODEL + d)
                            plsc.store_scatter(obufs[b], [idx], vec)

                    pltpu.async_copy(
                        obufs[b],
                        scr_hbm.at[
                            pl.ds(block_id(i) * (VBLOCK * D_MODEL), VBLOCK * D_MODEL)
                        ],
                        osems[b],
                    )

                    @pl.when(block_id(i + LEAD) < N_FULL_BLOCKS)
                    def _():
                        fire_load(i + LEAD, (b + LEAD) % NBUF)

            return carry

        lax.fori_loop(0, (n_steps + NBUF - 1) // NBUF, ring_body, 0)
        for b in range(NBUF):
            pltpu.make_async_copy(
                obufs[b], scr_hbm.at[pl.ds(0, VBLOCK * D_MODEL)], osems[b]
            ).wait()

        @pl.when(wid == NUM_WORKERS - 1)
        def _():
            pltpu.sync_copy(tail_hbm, obufs[0].at[pl.ds(0, TAIL_V * D_MODEL)])
            pltpu.sync_copy(
                obufs[0].at[pl.ds(0, TAIL_V * D_MODEL)],
                scr_hbm.at[pl.ds(N_FULL_BLOCKS * VBLOCK * D_MODEL, TAIL_V * D_MODEL)],
            )

    return k(tt, tail_scr)


@jax.jit
def _gather(xg, scr2):
    mesh = plsc.VectorSubcoreMesh(core_axis_name="c", subcore_axis_name="s")
    n_seq = xg.shape[1]
    n_dt = D_MODEL // 8

    @functools.partial(
        pl.kernel,
        mesh=mesh,
        out_type=jax.ShapeDtypeStruct((n_seq, n_dt, NUM_WORKERS, 1024), jnp.float32),
        scratch_types=[
            pltpu.VMEM((n_seq, GROUP), jnp.int32),
            tuple(pltpu.VMEM((GROUP, D_MODEL), jnp.float32) for _ in range(NBUF)),
            tuple(pltpu.VMEM((GROUP * D_MODEL,), jnp.float32) for _ in range(NBUF)),
            tuple(pltpu.SemaphoreType.DMA for _ in range(NBUF)),
            tuple(pltpu.SemaphoreType.DMA for _ in range(NBUF)),
        ],
        compiler_params=pltpu.CompilerParams(
            use_tc_tiling_on_sc=False, needs_layout_passes=False
        ),
    )
    def k(x_hbm, tab_hbm, out_hbm, idx_v, gbufs, obufs, gsems, ssems):
        wid = lax.axis_index("s") * NUM_CORES + lax.axis_index("c")
        iota = _iota16()
        pat = iota * GROUP
        pltpu.sync_copy(x_hbm.at[wid], idx_v)

        def fire_gather(s, b):
            pltpu.async_copy(tab_hbm.at[idx_v.at[s]], gbufs[b], gsems[b])

        def fire_store(s, b):
            for dt in range(n_dt):
                pltpu.async_copy(
                    obufs[b].at[pl.ds(dt * 1024, 1024)],
                    out_hbm.at[s, dt, wid],
                    ssems[b],
                )

        def wait_store(b):
            for dt in range(n_dt):
                pltpu.make_async_copy(
                    obufs[b].at[pl.ds(dt * 1024, 1024)],
                    out_hbm.at[0, 0, 0],
                    ssems[b],
                ).wait()

        for s0 in range(LEAD):
            fire_gather(s0, s0)

        def ring_body(p, carry):
            for b in range(NBUF):
                s = NBUF * p + b

                @pl.when(s < n_seq)
                def _():
                    pltpu.make_async_copy(
                        tab_hbm.at[idx_v.at[0]], gbufs[b], gsems[b]
                    ).wait()

                    @pl.when(s >= NBUF)
                    def _():
                        wait_store(b)

                    @plsc.parallel_loop(0, GROUP, unroll=4)
                    def _(bi):
                        for kk in range(D_MODEL // LANES):
                            vec = gbufs[b][bi, pl.ds(kk * LANES, LANES)]
                            idx = pat + (kk * LANES * GROUP + bi)
                            plsc.store_scatter(obufs[b], [idx], vec)

                    fire_store(s, b)

                    @pl.when(s + LEAD < n_seq)
                    def _():
                        fire_gather(s + LEAD, (b + LEAD) % NBUF)

            return carry

        lax.fori_loop(0, (n_seq + NBUF - 1) // NBUF, ring_body, 0)
        for b in range(NBUF):
            wait_store(b)

    return k(xg, scr2)


def kernel(x, embedding_weight):
    batch, seq = x.shape
    tt = jnp.transpose(embedding_weight)
    tail_scr = (jnp.transpose(tt[:, N_FULL_BLOCKS * VBLOCK :]) * SCALE).reshape(
        TAIL_V * D_MODEL
    )
    scr = _format(tt, tail_scr)
    scr2 = scr.reshape(VOCAB, D_MODEL)
    xg = (
        jnp.transpose(x.astype(jnp.int32))
        .reshape(seq, NUM_WORKERS, GROUP)
        .transpose(1, 0, 2)
    )
    out4 = _gather(xg, scr2)
    out5 = out4.reshape(seq, 8, NUM_WORKERS, 8, GROUP)
    return jnp.transpose(out5, (2, 4, 0, 1, 3)).reshape(batch, seq, D_MODEL)


# speedup vs baseline: 1.8997x; 1.8047x over previous
import functools

import jax
import jax.numpy as jnp
from jax import lax
from jax.experimental import pallas as pl
from jax.experimental.pallas import tpu as pltpu
from jax.experimental.pallas import tpu_sc as plsc

VOCAB = 1000000
D_MODEL = 64
LANES = 16
NUM_CORES = 2
NUM_SUBCORES = 16
NUM_WORKERS = NUM_CORES * NUM_SUBCORES
VBLOCK = 256
N_FULL_BLOCKS = VOCAB // VBLOCK
TAIL_V = VOCAB - N_FULL_BLOCKS * VBLOCK
GROUP = 128
SCALE = 8.0
NBUF = 3
LEAD = 2


def _iota16():
    return lax.iota(jnp.int32, LANES)


@jax.jit
def _format(tt, tail_scr):
    mesh = plsc.VectorSubcoreMesh(core_axis_name="c", subcore_axis_name="s")
    n_steps = (N_FULL_BLOCKS + NUM_WORKERS - 1) // NUM_WORKERS

    @functools.partial(
        pl.kernel,
        mesh=mesh,
        out_type=jax.ShapeDtypeStruct((VOCAB * D_MODEL,), jnp.float32),
        scratch_types=[
            tuple(pltpu.VMEM((D_MODEL, VBLOCK), jnp.float32) for _ in range(NBUF)),
            tuple(pltpu.VMEM((VBLOCK * D_MODEL,), jnp.float32) for _ in range(NBUF)),
            tuple(pltpu.SemaphoreType.DMA for _ in range(NBUF)),
            tuple(pltpu.SemaphoreType.DMA for _ in range(NBUF)),
        ],
        compiler_params=pltpu.CompilerParams(
            use_tc_tiling_on_sc=True, needs_layout_passes=False
        ),
    )
    def k(tt_hbm, tail_hbm, scr_hbm, tbufs, obufs, isems, osems):
        wid = lax.axis_index("s") * NUM_CORES + lax.axis_index("c")
        iota = _iota16()
        pat = iota * D_MODEL

        def block_id(i):
            return wid + NUM_WORKERS * i

        def fire_load(i, b):
            pltpu.async_copy(
                tt_hbm.at[pl.ds(0, D_MODEL), pl.ds(block_id(i) * VBLOCK, VBLOCK)],
                tbufs[b],
                isems[b],
            )

        for i0 in range(LEAD):
            fire_load(i0, i0)

        def ring_body(p, carry):
            for b in range(NBUF):
                i = NBUF * p + b

                @pl.when(block_id(i) < N_FULL_BLOCKS)
                def _():
                    pltpu.make_async_copy(
                        tt_hbm.at[pl.ds(0, D_MODEL), pl.ds(0, VBLOCK)],
                        tbufs[b],
                        isems[b],
                    ).wait()

                    @pl.when(i >= NBUF)
                    def _():
                        pltpu.make_async_copy(
                            obufs[b], scr_hbm.at[pl.ds(0, VBLOCK * D_MODEL)], osems[b]
                        ).wait()

                    @plsc.parallel_loop(0, LANES, unroll=1)
                    def _(r):
                        w = (iota + r) & (LANES - 1)
                        dst_pre = w * D_MODEL + iota
                        for dc in range(D_MODEL // LANES):
                            d_idx = iota + (dc * LANES)
                            for vc in range(VBLOCK // LANES):
                                v_idx = w + (vc * LANES)
                                vec = plsc.load_gather(tbufs[b], [d_idx, v_idx])
                                dst = dst_pre + (vc * LANES * D_MODEL + dc * LANES)
                                plsc.store_scatter(obufs[b], [dst], vec * SCALE)

                    pltpu.async_copy(
                        obufs[b],
                        scr_hbm.at[
                            pl.ds(block_id(i) * (VBLOCK * D_MODEL), VBLOCK * D_MODEL)
                        ],
                        osems[b],
                    )

                    @pl.when(block_id(i + LEAD) < N_FULL_BLOCKS)
                    def _():
                        fire_load(i + LEAD, (b + LEAD) % NBUF)

            return carry

        lax.fori_loop(0, (n_steps + NBUF - 1) // NBUF, ring_body, 0)
        for b in range(NBUF):
            pltpu.make_async_copy(
                obufs[b], scr_hbm.at[pl.ds(0, VBLOCK * D_MODEL)], osems[b]
            ).wait()

        @pl.when(wid == NUM_WORKERS - 1)
        def _():
            pltpu.sync_copy(tail_hbm, obufs[0].at[pl.ds(0, TAIL_V * D_MODEL)])
            pltpu.sync_copy(
                obufs[0].at[pl.ds(0, TAIL_V * D_MODEL)],
                scr_hbm.at[pl.ds(N_FULL_BLOCKS * VBLOCK * D_MODEL, TAIL_V * D_MODEL)],
            )

    return k(tt, tail_scr)


@jax.jit
def _gather(xg, scr2):
    mesh = plsc.VectorSubcoreMesh(core_axis_name="c", subcore_axis_name="s")
    n_seq = xg.shape[1]
    n_dt = D_MODEL // 8

    @functools.partial(
        pl.kernel,
        mesh=mesh,
        out_type=jax.ShapeDtypeStruct((n_seq, n_dt, NUM_WORKERS, 1024), jnp.float32),
        scratch_types=[
            pltpu.VMEM((n_seq, GROUP), jnp.int32),
            tuple(pltpu.VMEM((GROUP, D_MODEL), jnp.float32) for _ in range(NBUF)),
            tuple(pltpu.VMEM((GROUP * D_MODEL,), jnp.float32) for _ in range(NBUF)),
            tuple(pltpu.SemaphoreType.DMA for _ in range(NBUF)),
            tuple(pltpu.SemaphoreType.DMA for _ in range(NBUF)),
        ],
        compiler_params=pltpu.CompilerParams(
            use_tc_tiling_on_sc=False, needs_layout_passes=False
        ),
    )
    def k(x_hbm, tab_hbm, out_hbm, idx_v, gbufs, obufs, gsems, ssems):
        wid = lax.axis_index("s") * NUM_CORES + lax.axis_index("c")
        iota = _iota16()
        pat = iota * GROUP
        pltpu.sync_copy(x_hbm.at[wid], idx_v)

        def fire_gather(s, b):
            pltpu.async_copy(tab_hbm.at[idx_v.at[s]], gbufs[b], gsems[b])

        def fire_store(s, b):
            for dt in range(n_dt):
                pltpu.async_copy(
                    obufs[b].at[pl.ds(dt * 1024, 1024)],
                    out_hbm.at[s, dt, wid],
                    ssems[b],
                )

        def wait_store(b):
            for dt in range(n_dt):
                pltpu.make_async_copy(
                    obufs[b].at[pl.ds(dt * 1024, 1024)],
                    out_hbm.at[0, 0, 0],
                    ssems[b],
                ).wait()

        for s0 in range(LEAD):
            fire_gather(s0, s0)

        def ring_body(p, carry):
            for b in range(NBUF):
                s = NBUF * p + b

                @pl.when(s < n_seq)
                def _():
                    pltpu.make_async_copy(
                        tab_hbm.at[idx_v.at[0]], gbufs[b], gsems[b]
                    ).wait()

                    @pl.when(s >= NBUF)
                    def _():
                        wait_store(b)

                    @plsc.parallel_loop(0, LANES, unroll=1)
                    def _(r):
                        w = (iota + r) & (LANES - 1)
                        dst_pre = w * GROUP + iota
                        for dc in range(D_MODEL // LANES):
                            for bic in range(GROUP // LANES):
                                bi_idx = iota + (bic * LANES)
                                d_idx = w + (dc * LANES)
                                vec = plsc.load_gather(gbufs[b], [bi_idx, d_idx])
                                dst = dst_pre + (dc * LANES * GROUP + bic * LANES)
                                plsc.store_scatter(obufs[b], [dst], vec)

                    fire_store(s, b)

                    @pl.when(s + LEAD < n_seq)
                    def _():
                        fire_gather(s + LEAD, (b + LEAD) % NBUF)

            return carry

        lax.fori_loop(0, (n_seq + NBUF - 1) // NBUF, ring_body, 0)
        for b in range(NBUF):
            wait_store(b)

    return k(xg, scr2)


def kernel(x, embedding_weight):
    batch, seq = x.shape
    tt = jnp.transpose(embedding_weight)
    tail_scr = (jnp.transpose(tt[:, N_FULL_BLOCKS * VBLOCK :]) * SCALE).reshape(
        TAIL_V * D_MODEL
    )
    scr = _format(tt, tail_scr)
    scr2 = scr.reshape(VOCAB, D_MODEL)
    xg = (
        jnp.transpose(x.astype(jnp.int32))
        .reshape(seq, NUM_WORKERS, GROUP)
        .transpose(1, 0, 2)
    )
    out4 = _gather(xg, scr2)
    out5 = out4.reshape(seq, 8, NUM_WORKERS, 8, GROUP)
    return jnp.transpose(out5, (2, 4, 0, 1, 3)).reshape(batch, seq, D_MODEL)


# speedup vs baseline: 2.1163x; 1.1140x over previous
import functools

import jax
import jax.numpy as jnp
from jax import lax
from jax.experimental import pallas as pl
from jax.experimental.pallas import tpu as pltpu
from jax.experimental.pallas import tpu_sc as plsc

VOCAB = 1000000
D_MODEL = 64
LANES = 16
NUM_CORES = 2
NUM_SUBCORES = 16
NUM_WORKERS = NUM_CORES * NUM_SUBCORES
VBLOCK = 256
N_FULL_BLOCKS = VOCAB // VBLOCK
TAIL_V = VOCAB - N_FULL_BLOCKS * VBLOCK
GROUP = 128
SCALE = 8.0
NBUF = 3
LEAD = 2


def _iota16():
    return lax.iota(jnp.int32, LANES)


@jax.jit
def _format(tt, tail_scr):
    mesh = plsc.VectorSubcoreMesh(core_axis_name="c", subcore_axis_name="s")
    n_steps = (N_FULL_BLOCKS + NUM_WORKERS - 1) // NUM_WORKERS

    @functools.partial(
        pl.kernel,
        mesh=mesh,
        out_type=jax.ShapeDtypeStruct((VOCAB * D_MODEL,), jnp.float32),
        scratch_types=[
            tuple(pltpu.VMEM((D_MODEL, VBLOCK), jnp.float32) for _ in range(NBUF)),
            tuple(pltpu.VMEM((VBLOCK * D_MODEL,), jnp.float32) for _ in range(NBUF)),
            tuple(pltpu.SemaphoreType.DMA for _ in range(NBUF)),
            tuple(pltpu.SemaphoreType.DMA for _ in range(NBUF)),
        ],
        compiler_params=pltpu.CompilerParams(
            use_tc_tiling_on_sc=True, needs_layout_passes=False
        ),
    )
    def k(tt_hbm, tail_hbm, scr_hbm, tbufs, obufs, isems, osems):
        wid = lax.axis_index("s") * NUM_CORES + lax.axis_index("c")
        iota = _iota16()
        pat = iota * D_MODEL

        def block_id(i):
            return wid + NUM_WORKERS * i

        def fire_load(i, b):
            pltpu.async_copy(
                tt_hbm.at[pl.ds(0, D_MODEL), pl.ds(block_id(i) * VBLOCK, VBLOCK)],
                tbufs[b],
                isems[b],
            )

        for i0 in range(LEAD):
            fire_load(i0, i0)

        def ring_body(p, carry):
            for b in range(NBUF):
                i = NBUF * p + b

                @pl.when(block_id(i) < N_FULL_BLOCKS)
                def _():
                    pltpu.make_async_copy(
                        tt_hbm.at[pl.ds(0, D_MODEL), pl.ds(0, VBLOCK)],
                        tbufs[b],
                        isems[b],
                    ).wait()

                    @pl.when(i >= NBUF)
                    def _():
                        pltpu.make_async_copy(
                            obufs[b], scr_hbm.at[pl.ds(0, VBLOCK * D_MODEL)], osems[b]
                        ).wait()

                    @plsc.parallel_loop(0, LANES, unroll=2)
                    def _(r):
                        w = (iota + r) & (LANES - 1)
                        dst_pre = w * D_MODEL + iota
                        for dc in range(D_MODEL // LANES):
                            d_idx = iota + (dc * LANES)
                            for vc in range(VBLOCK // LANES):
                                v_idx = w + (vc * LANES)
                                vec = plsc.load_gather(tbufs[b], [d_idx, v_idx])
                                dst = dst_pre + (vc * LANES * D_MODEL + dc * LANES)
                                plsc.store_scatter(obufs[b], [dst], vec)

                    pltpu.async_copy(
                        obufs[b],
                        scr_hbm.at[
                            pl.ds(block_id(i) * (VBLOCK * D_MODEL), VBLOCK * D_MODEL)
                        ],
                        osems[b],
                    )

                    @pl.when(block_id(i + LEAD) < N_FULL_BLOCKS)
                    def _():
                        fire_load(i + LEAD, (b + LEAD) % NBUF)

            return carry

        lax.fori_loop(0, (n_steps + NBUF - 1) // NBUF, ring_body, 0)
        for b in range(NBUF):
            pltpu.make_async_copy(
                obufs[b], scr_hbm.at[pl.ds(0, VBLOCK * D_MODEL)], osems[b]
            ).wait()

        @pl.when(wid == NUM_WORKERS - 1)
        def _():
            pltpu.sync_copy(tail_hbm, obufs[0].at[pl.ds(0, TAIL_V * D_MODEL)])
            pltpu.sync_copy(
                obufs[0].at[pl.ds(0, TAIL_V * D_MODEL)],
                scr_hbm.at[pl.ds(N_FULL_BLOCKS * VBLOCK * D_MODEL, TAIL_V * D_MODEL)],
            )

    return k(tt, tail_scr)


@jax.jit
def _gather(xg, scr2):
    mesh = plsc.VectorSubcoreMesh(core_axis_name="c", subcore_axis_name="s")
    n_seq = xg.shape[1]
    n_dt = D_MODEL // 8

    @functools.partial(
        pl.kernel,
        mesh=mesh,
        out_type=jax.ShapeDtypeStruct((n_seq, n_dt, NUM_WORKERS, 1024), jnp.float32),
        scratch_types=[
            pltpu.VMEM((n_seq, GROUP), jnp.int32),
            tuple(pltpu.VMEM((GROUP, D_MODEL), jnp.float32) for _ in range(NBUF)),
            tuple(pltpu.VMEM((GROUP * D_MODEL,), jnp.float32) for _ in range(NBUF)),
            tuple(pltpu.SemaphoreType.DMA for _ in range(NBUF)),
            tuple(pltpu.SemaphoreType.DMA for _ in range(NBUF)),
        ],
        compiler_params=pltpu.CompilerParams(
            use_tc_tiling_on_sc=False, needs_layout_passes=False
        ),
    )
    def k(x_hbm, tab_hbm, out_hbm, idx_v, gbufs, obufs, gsems, ssems):
        wid = lax.axis_index("s") * NUM_CORES + lax.axis_index("c")
        iota = _iota16()
        pat = iota * GROUP
        pltpu.sync_copy(x_hbm.at[wid], idx_v)

        def fire_gather(s, b):
            pltpu.async_copy(tab_hbm.at[idx_v.at[s]], gbufs[b], gsems[b])

        def fire_store(s, b):
            for dt in range(n_dt):
                pltpu.async_copy(
                    obufs[b].at[pl.ds(dt * 1024, 1024)],
                    out_hbm.at[s, dt, wid],
                    ssems[b],
                )

        def wait_store(b):
            for dt in range(n_dt):
                pltpu.make_async_copy(
                    obufs[b].at[pl.ds(dt * 1024, 1024)],
                    out_hbm.at[0, 0, 0],
                    ssems[b],
                ).wait()

        for s0 in range(LEAD):
            fire_gather(s0, s0)

        def ring_body(p, carry):
            for b in range(NBUF):
                s = NBUF * p + b

                @pl.when(s < n_seq)
                def _():
                    pltpu.make_async_copy(
                        tab_hbm.at[idx_v.at[0]], gbufs[b], gsems[b]
                    ).wait()

                    @pl.when(s >= NBUF)
                    def _():
                        wait_store(b)

                    @plsc.parallel_loop(0, LANES, unroll=1)
                    def _(r):
                        w = (iota + r) & (LANES - 1)
                        dst_pre = w * GROUP + iota
                        for dc in range(D_MODEL // LANES):
                            for bic in range(GROUP // LANES):
                                bi_idx = iota + (bic * LANES)
                                d_idx = w + (dc * LANES)
                                vec = plsc.load_gather(gbufs[b], [bi_idx, d_idx])
                                dst = dst_pre + (dc * LANES * GROUP + bic * LANES)
                                plsc.store_scatter(obufs[b], [dst], vec * SCALE)

                    fire_store(s, b)

                    @pl.when(s + LEAD < n_seq)
                    def _():
                        fire_gather(s + LEAD, (b + LEAD) % NBUF)

            return carry

        lax.fori_loop(0, (n_seq + NBUF - 1) // NBUF, ring_body, 0)
        for b in range(NBUF):
            wait_store(b)

    return k(xg, scr2)


def kernel(x, embedding_weight):
    batch, seq = x.shape
    tt = jnp.transpose(embedding_weight)
    tail_scr = jnp.transpose(tt[:, N_FULL_BLOCKS * VBLOCK :]).reshape(
        TAIL_V * D_MODEL
    )
    scr = _format(tt, tail_scr)
    scr2 = scr.reshape(VOCAB, D_MODEL)
    xg = (
        jnp.transpose(x.astype(jnp.int32))
        .reshape(seq, NUM_WORKERS, GROUP)
        .transpose(1, 0, 2)
    )
    out4 = _gather(xg, scr2)
    out5 = out4.reshape(seq, 8, NUM_WORKERS, 8, GROUP)
    return jnp.transpose(out5, (2, 4, 0, 1, 3)).reshape(batch, seq, D_MODEL)


# speedup vs baseline: 2.1459x; 1.0140x over previous
import functools

import jax
import jax.numpy as jnp
from jax import lax
from jax.experimental import pallas as pl
from jax.experimental.pallas import tpu as pltpu
from jax.experimental.pallas import tpu_sc as plsc

VOCAB = 1000000
D_MODEL = 64
LANES = 16
NUM_CORES = 2
NUM_SUBCORES = 16
NUM_WORKERS = NUM_CORES * NUM_SUBCORES
VBLOCK = 256
N_FULL_BLOCKS = VOCAB // VBLOCK
TAIL_V = VOCAB - N_FULL_BLOCKS * VBLOCK
GROUP = 128
SCALE = 8.0
NBUF = 3
LEAD = 2


def _iota16():
    return lax.iota(jnp.int32, LANES)


@jax.jit
def _format(tt, tail_scr):
    mesh = plsc.VectorSubcoreMesh(core_axis_name="c", subcore_axis_name="s")
    n_steps = (N_FULL_BLOCKS + NUM_WORKERS - 1) // NUM_WORKERS

    @functools.partial(
        pl.kernel,
        mesh=mesh,
        out_type=jax.ShapeDtypeStruct((VOCAB * D_MODEL,), jnp.float32),
        scratch_types=[
            tuple(pltpu.VMEM((D_MODEL, VBLOCK), jnp.float32) for _ in range(NBUF)),
            tuple(pltpu.VMEM((VBLOCK * D_MODEL,), jnp.float32) for _ in range(NBUF)),
            tuple(pltpu.SemaphoreType.DMA for _ in range(NBUF)),
            tuple(pltpu.SemaphoreType.DMA for _ in range(NBUF)),
        ],
        compiler_params=pltpu.CompilerParams(
            use_tc_tiling_on_sc=True, needs_layout_passes=False
        ),
    )
    def k(tt_hbm, tail_hbm, scr_hbm, tbufs, obufs, isems, osems):
        wid = lax.axis_index("s") * NUM_CORES + lax.axis_index("c")
        iota = _iota16()
        pat = iota * D_MODEL

        def block_id(i):
            return wid + NUM_WORKERS * i

        def fire_load(i, b):
            pltpu.async_copy(
                tt_hbm.at[pl.ds(0, D_MODEL), pl.ds(block_id(i) * VBLOCK, VBLOCK)],
                tbufs[b],
                isems[b],
            )

        for i0 in range(LEAD):
            fire_load(i0, i0)

        def ring_body(p, carry):
            for b in range(NBUF):
                i = NBUF * p + b

                @pl.when(block_id(i) < N_FULL_BLOCKS)
                def _():
                    pltpu.make_async_copy(
                        tt_hbm.at[pl.ds(0, D_MODEL), pl.ds(0, VBLOCK)],
                        tbufs[b],
                        isems[b],
                    ).wait()

                    @pl.when(i >= NBUF)
                    def _():
                        pltpu.make_async_copy(
                            obufs[b], scr_hbm.at[pl.ds(0, VBLOCK * D_MODEL)], osems[b]
                        ).wait()

                    @plsc.parallel_loop(0, LANES, unroll=2)
                    def _(r):
                        w = (iota + r) & (LANES - 1)
                        dst_pre = w * D_MODEL + iota
                        for dc in range(D_MODEL // LANES):
                            d_idx = iota + (dc * LANES)
                            for vc in range(VBLOCK // LANES):
                                v_idx = w + (vc * LANES)
                                vec = plsc.load_gather(tbufs[b], [d_idx, v_idx])
                                dst = dst_pre + (vc * LANES * D_MODEL + dc * LANES)
                                plsc.store_scatter(obufs[b], [dst], vec)

                    pltpu.async_copy(
                        obufs[b],
                        scr_hbm.at[
                            pl.ds(block_id(i) * (VBLOCK * D_MODEL), VBLOCK * D_MODEL)
                        ],
                        osems[b],
                    )

                    @pl.when(block_id(i + LEAD) < N_FULL_BLOCKS)
                    def _():
                        fire_load(i + LEAD, (b + LEAD) % NBUF)

            return carry

        lax.fori_loop(0, (n_steps + NBUF - 1) // NBUF, ring_body, 0)
        for b in range(NBUF):
            pltpu.make_async_copy(
                obufs[b], scr_hbm.at[pl.ds(0, VBLOCK * D_MODEL)], osems[b]
            ).wait()

        @pl.when(wid == NUM_WORKERS - 1)
        def _():
            pltpu.sync_copy(tail_hbm, obufs[0].at[pl.ds(0, TAIL_V * D_MODEL)])
            pltpu.sync_copy(
                obufs[0].at[pl.ds(0, TAIL_V * D_MODEL)],
                scr_hbm.at[pl.ds(N_FULL_BLOCKS * VBLOCK * D_MODEL, TAIL_V * D_MODEL)],
            )

    return k(tt, tail_scr)


@jax.jit
def _gather(xg, scr2):
    mesh = plsc.VectorSubcoreMesh(core_axis_name="c", subcore_axis_name="s")
    n_seq = xg.shape[1]
    n_dt = D_MODEL // 8

    @functools.partial(
        pl.kernel,
        mesh=mesh,
        out_type=jax.ShapeDtypeStruct((n_seq, n_dt, NUM_WORKERS, 1024), jnp.float32),
        scratch_types=[
            pltpu.VMEM((n_seq, GROUP), jnp.int32),
            tuple(pltpu.VMEM((GROUP, D_MODEL), jnp.float32) for _ in range(NBUF)),
            tuple(pltpu.VMEM((GROUP * D_MODEL,), jnp.float32) for _ in range(NBUF)),
            tuple(pltpu.SemaphoreType.DMA for _ in range(NBUF)),
            tuple(pltpu.SemaphoreType.DMA for _ in range(NBUF)),
        ],
        compiler_params=pltpu.CompilerParams(
            use_tc_tiling_on_sc=False, needs_layout_passes=False
        ),
    )
    def k(x_hbm, tab_hbm, out_hbm, idx_v, gbufs, obufs, gsems, ssems):
        wid = lax.axis_index("s") * NUM_CORES + lax.axis_index("c")
        iota = _iota16()
        pat = iota * GROUP
        pltpu.sync_copy(x_hbm.at[wid], idx_v)

        def fire_gather(s, b):
            pltpu.async_copy(tab_hbm.at[idx_v.at[s]], gbufs[b], gsems[b])

        def fire_store(s, b):
            for dt in range(n_dt):
                pltpu.async_copy(
                    obufs[b].at[pl.ds(dt * 1024, 1024)],
                    out_hbm.at[s, dt, wid],
                    ssems[b],
                )

        def wait_store(b):
            for dt in range(n_dt):
                pltpu.make_async_copy(
                    obufs[b].at[pl.ds(dt * 1024, 1024)],
                    out_hbm.at[0, 0, 0],
                    ssems[b],
                ).wait()

        for s0 in range(LEAD):
            fire_gather(s0, s0)

        def ring_body(p, carry):
            for b in range(NBUF):
                s = NBUF * p + b

                @pl.when(s < n_seq)
                def _():
                    pltpu.make_async_copy(
                        tab_hbm.at[idx_v.at[0]], gbufs[b], gsems[b]
                    ).wait()

                    @pl.when(s >= NBUF)
                    def _():
                        wait_store(b)

                    @plsc.parallel_loop(0, LANES, unroll=2)
                    def _(r):
                        w = (iota + r) & (LANES - 1)
                        dst_pre = w * GROUP + iota
                        for dc in range(D_MODEL // LANES):
                            for bic in range(GROUP // LANES):
                                bi_idx = iota + (bic * LANES)
                                d_idx = w + (dc * LANES)
                                vec = plsc.load_gather(gbufs[b], [bi_idx, d_idx])
                                dst = dst_pre + (dc * LANES * GROUP + bic * LANES)
                                plsc.store_scatter(obufs[b], [dst], vec * SCALE)

                    fire_store(s, b)

                    @pl.when(s + LEAD < n_seq)
                    def _():
                        fire_gather(s + LEAD, (b + LEAD) % NBUF)

            return carry

        lax.fori_loop(0, (n_seq + NBUF - 1) // NBUF, ring_body, 0)
        for b in range(NBUF):
            wait_store(b)

    return k(xg, scr2)


def kernel(x, embedding_weight):
    batch, seq = x.shape
    tt = jnp.transpose(embedding_weight)
    tail_scr = jnp.transpose(tt[:, N_FULL_BLOCKS * VBLOCK :]).reshape(
        TAIL_V * D_MODEL
    )
    scr = _format(tt, tail_scr)
    scr2 = scr.reshape(VOCAB, D_MODEL)
    xg = (
        jnp.transpose(x.astype(jnp.int32))
        .reshape(seq, NUM_WORKERS, GROUP)
        .transpose(1, 0, 2)
    )
    out4 = _gather(xg, scr2)
    out5 = out4.reshape(seq, 8, NUM_WORKERS, 8, GROUP)
    return jnp.transpose(out5, (2, 4, 0, 1, 3)).reshape(batch, seq, D_MODEL)


# speedup vs baseline: 2.9488x; 1.3741x over previous
import functools

import jax
import jax.numpy as jnp
from jax import lax
from jax.experimental import pallas as pl
from jax.experimental.pallas import tpu as pltpu
from jax.experimental.pallas import tpu_sc as plsc

VOCAB = 1000000
D_MODEL = 64
LANES = 16
NUM_CORES = 2
NUM_SUBCORES = 16
NUM_WORKERS = NUM_CORES * NUM_SUBCORES
VBLOCK = 256
N_FULL_BLOCKS = VOCAB // VBLOCK
TAIL_V = VOCAB - N_FULL_BLOCKS * VBLOCK
GROUP = 128
SCALE = 8.0
NBUF = 3
LEAD = 2


def _iota16():
    return lax.iota(jnp.int32, LANES)


@jax.jit
def _format(tt, tail_scr):
    mesh = plsc.VectorSubcoreMesh(core_axis_name="c", subcore_axis_name="s")
    n_steps = (N_FULL_BLOCKS + NUM_WORKERS - 1) // NUM_WORKERS

    @functools.partial(
        pl.kernel,
        mesh=mesh,
        out_type=jax.ShapeDtypeStruct((VOCAB * D_MODEL,), jnp.float32),
        scratch_types=[
            tuple(pltpu.VMEM((D_MODEL, VBLOCK), jnp.float32) for _ in range(NBUF)),
            tuple(pltpu.VMEM((VBLOCK * D_MODEL,), jnp.float32) for _ in range(NBUF)),
            tuple(pltpu.SemaphoreType.DMA for _ in range(NBUF)),
            tuple(pltpu.SemaphoreType.DMA for _ in range(NBUF)),
        ],
        compiler_params=pltpu.CompilerParams(
            use_tc_tiling_on_sc=True, needs_layout_passes=False
        ),
    )
    def k(tt_hbm, tail_hbm, scr_hbm, tbufs, obufs, isems, osems):
        wid = lax.axis_index("s") * NUM_CORES + lax.axis_index("c")
        iota = _iota16()
        pat = iota * D_MODEL

        def block_id(i):
            return wid + NUM_WORKERS * i

        def fire_load(i, b):
            pltpu.async_copy(
                tt_hbm.at[pl.ds(0, D_MODEL), pl.ds(block_id(i) * VBLOCK, VBLOCK)],
                tbufs[b],
                isems[b],
            )

        for i0 in range(LEAD):
            fire_load(i0, i0)

        def ring_body(p, carry):
            for b in range(NBUF):
                i = NBUF * p + b

                @pl.when(block_id(i) < N_FULL_BLOCKS)
                def _():
                    pltpu.make_async_copy(
                        tt_hbm.at[pl.ds(0, D_MODEL), pl.ds(0, VBLOCK)],
                        tbufs[b],
                        isems[b],
                    ).wait()

                    @pl.when(i >= NBUF)
                    def _():
                        pltpu.make_async_copy(
                            obufs[b], scr_hbm.at[pl.ds(0, VBLOCK * D_MODEL)], osems[b]
                        ).wait()

                    @plsc.parallel_loop(0, LANES, unroll=4)
                    def _(r):
                        w = (iota + r) & (LANES - 1)
                        dst_pre = w * D_MODEL + iota
                        for dc in range(D_MODEL // LANES):
                            d_idx = iota + (dc * LANES)
                            for vc in range(VBLOCK // LANES):
                                v_idx = w + (vc * LANES)
                                vec = plsc.load_gather(tbufs[b], [d_idx, v_idx])
                                dst = dst_pre + (vc * LANES * D_MODEL + dc * LANES)
                                plsc.store_scatter(obufs[b], [dst], vec)

                    pltpu.async_copy(
                        obufs[b],
                        scr_hbm.at[
                            pl.ds(block_id(i) * (VBLOCK * D_MODEL), VBLOCK * D_MODEL)
                        ],
                        osems[b],
                    )

                    @pl.when(block_id(i + LEAD) < N_FULL_BLOCKS)
                    def _():
                        fire_load(i + LEAD, (b + LEAD) % NBUF)

            return carry

        lax.fori_loop(0, (n_steps + NBUF - 1) // NBUF, ring_body, 0)
        for b in range(NBUF):
            pltpu.make_async_copy(
                obufs[b], scr_hbm.at[pl.ds(0, VBLOCK * D_MODEL)], osems[b]
            ).wait()

        @pl.when(wid == NUM_WORKERS - 1)
        def _():
            pltpu.sync_copy(tail_hbm, obufs[0].at[pl.ds(0, TAIL_V * D_MODEL)])
            pltpu.sync_copy(
                obufs[0].at[pl.ds(0, TAIL_V * D_MODEL)],
                scr_hbm.at[pl.ds(N_FULL_BLOCKS * VBLOCK * D_MODEL, TAIL_V * D_MODEL)],
            )

    return k(tt, tail_scr)


@jax.jit
def _gather(xg, scr2):
    mesh = plsc.VectorSubcoreMesh(core_axis_name="c", subcore_axis_name="s")
    n_seq = xg.shape[1]
    n_dt = D_MODEL // 8

    @functools.partial(
        pl.kernel,
        mesh=mesh,
        out_type=jax.ShapeDtypeStruct((n_seq, n_dt, NUM_WORKERS, 1024), jnp.float32),
        scratch_types=[
            pltpu.VMEM((n_seq, GROUP), jnp.int32),
            tuple(pltpu.VMEM((GROUP, D_MODEL), jnp.float32) for _ in range(NBUF)),
            tuple(pltpu.VMEM((GROUP * D_MODEL,), jnp.float32) for _ in range(NBUF)),
            tuple(pltpu.SemaphoreType.DMA for _ in range(NBUF)),
            tuple(pltpu.SemaphoreType.DMA for _ in range(NBUF)),
        ],
        compiler_params=pltpu.CompilerParams(
            use_tc_tiling_on_sc=False, needs_layout_passes=False
        ),
    )
    def k(x_hbm, tab_hbm, out_hbm, idx_v, gbufs, obufs, gsems, ssems):
        wid = lax.axis_index("s") * NUM_CORES + lax.axis_index("c")
        iota = _iota16()
        pat = iota * GROUP
        pltpu.sync_copy(x_hbm.at[wid], idx_v)

        def fire_gather(s, b):
            pltpu.async_copy(tab_hbm.at[idx_v.at[s]], gbufs[b], gsems[b])

        def fire_store(s, b):
            for dt in range(n_dt):
                pltpu.async_copy(
                    obufs[b].at[pl.ds(dt * 1024, 1024)],
                    out_hbm.at[s, dt, wid],
                    ssems[b],
                )

        def wait_store(b):
            for dt in range(n_dt):
                pltpu.make_async_copy(
                    obufs[b].at[pl.ds(dt * 1024, 1024)],
                    out_hbm.at[0, 0, 0],
                    ssems[b],
                ).wait()

        for s0 in range(LEAD):
            fire_gather(s0, s0)

        def ring_body(p, carry):
            for b in range(NBUF):
                s = NBUF * p + b

                @pl.when(s < n_seq)
                def _():
                    pltpu.make_async_copy(
                        tab_hbm.at[idx_v.at[0]], gbufs[b], gsems[b]
                    ).wait()

                    @pl.when(s >= NBUF)
                    def _():
                        wait_store(b)

                    @plsc.parallel_loop(0, LANES, unroll=2)
                    def _(r):
                        w = (iota + r) & (LANES - 1)
                        dst_pre = w * GROUP + iota
                        for dc in range(D_MODEL // LANES):
                            for bic in range(GROUP // LANES):
                                bi_idx = iota + (bic * LANES)
                                d_idx = w + (dc * LANES)
                                vec = plsc.load_gather(gbufs[b], [bi_idx, d_idx])
                                dst = dst_pre + (dc * LANES * GROUP + bic * LANES)
                                plsc.store_scatter(obufs[b], [dst], vec * SCALE)

                    fire_store(s, b)

                    @pl.when(s + LEAD < n_seq)
                    def _():
                        fire_gather(s + LEAD, (b + LEAD) % NBUF)

            return carry

        lax.fori_loop(0, (n_seq + NBUF - 1) // NBUF, ring_body, 0)
        for b in range(NBUF):
            wait_store(b)

    return k(xg, scr2)


def kernel(x, embedding_weight):
    batch, seq = x.shape
    tt = jnp.transpose(embedding_weight)
    tail_scr = jnp.transpose(tt[:, N_FULL_BLOCKS * VBLOCK :]).reshape(
        TAIL_V * D_MODEL
    )
    scr = _format(tt, tail_scr)
    scr2 = scr.reshape(VOCAB, D_MODEL)
    xg = (
        jnp.transpose(x.astype(jnp.int32))
        .reshape(seq, NUM_WORKERS, GROUP)
        .transpose(1, 0, 2)
    )
    out4 = _gather(xg, scr2)
    out5 = out4.reshape(seq, 8, NUM_WORKERS, 8, GROUP)
    return jnp.transpose(out5, (2, 4, 0, 1, 3)).reshape(batch, seq, D_MODEL)
